# baseline (speedup 1.0000x reference)
import functools

import jax
import jax.numpy as jnp
from jax import lax
from jax.experimental import pallas as pl
from jax.experimental.pallas import tpu as pltpu
from jax.experimental.pallas import tpu_sc as plsc

_NC = 2
_NS = 16
_NW = _NC * _NS
_LANES = 16
_CHUNK = 128
_ROWS_PER_BLOCK = 8
_WAVE = 128


def _mf_body(b_per_w, d_half, idx_u_hbm, idx_i_hbm, pu_hbm, pi_hbm, out_hbm,
             idx_u_v, idx_i_v, blk_u_v, blk_i_v, u_rows, i_rows, out_v, sem):
    wid = lax.axis_index("s") * _NC + lax.axis_index("c")
    base = wid * b_per_w
    n_waves = b_per_w // _WAVE

    cp_u = pltpu.async_copy(idx_u_hbm.at[pl.ds(base, b_per_w)], idx_u_v, sem)
    cp_i = pltpu.async_copy(idx_i_hbm.at[pl.ds(base, b_per_w)], idx_i_v, sem)
    cp_u.wait()
    cp_i.wait()

    def blockify(t, carry):
        sl = pl.ds(t * _LANES, _LANES)
        blk_u_v[sl] = jax.lax.shift_right_logical(idx_u_v[sl], 3)
        blk_i_v[sl] = jax.lax.shift_right_logical(idx_i_v[sl], 3)
        return carry

    lax.fori_loop(0, b_per_w // _LANES, blockify, None)

    def fire(w):
        copies = []
        for k in range(_WAVE // _CHUNK):
            sl = pl.ds(w * _WAVE + k * _CHUNK, _CHUNK)
            dsl = pl.ds(k * _CHUNK, _CHUNK)
            copies.append(pltpu.async_copy(
                pu_hbm.at[blk_u_v.at[sl]], u_rows.at[w % 2].at[dsl], sem))
            copies.append(pltpu.async_copy(
                pi_hbm.at[blk_i_v.at[sl]], i_rows.at[w % 2].at[dsl], sem))
        return copies

    def compute(w):
        ubuf = u_rows.at[w % 2]
        ibuf = i_rows.at[w % 2]

        def group(g, carry):
            sl = pl.ds(w * _WAVE + g * _LANES, _LANES)
            row = g * _LANES + lax.iota(jnp.int32, _LANES)
            off_u = jax.lax.shift_left(idx_u_v[sl] & 7, 4)
            off_i = jax.lax.shift_left(idx_i_v[sl] & 7, 4)
            acc = jnp.zeros((_LANES,), jnp.float32)
            for j in range(d_half):
                gu = plsc.load_gather(ubuf, [row, off_u + j])
                gi = plsc.load_gather(ibuf, [row, off_i + j])
                u0, u1 = plsc.unpack(plsc.bitcast(gu, jnp.bfloat16),
                                     format=plsc.PackFormat.INTERLEAVED)
                i0, i1 = plsc.unpack(plsc.bitcast(gi, jnp.bfloat16),
                                     format=plsc.PackFormat.INTERLEAVED)
                acc = acc + u0 * i0 + u1 * i1
            out_v[sl] = 1.0 / (1.0 + jnp.exp(-acc))
            return carry

        lax.fori_loop(0, _WAVE // _LANES, group, None)

    pending = fire(0)
    for w in range(n_waves):
        for cp in pending:
            cp.wait()
        if w + 1 < n_waves:
            pending = fire(w + 1)
        compute(w)

    pltpu.sync_copy(out_v, out_hbm.at[pl.ds(base, b_per_w)])


def _pack_table(embeds):
    n, d = embeds.shape
    rows = (n * d) // (2 * _CHUNK)
    return jax.lax.bitcast_convert_type(
        embeds.astype(jnp.bfloat16).reshape(rows, _CHUNK, 2), jnp.int32)


def kernel(idx_u, idx_i, embeds_u, embeds_i):
    batch = idx_u.shape[0]
    d_half = embeds_u.shape[1] // 2
    b_per_w = batch // _NW
    mesh = plsc.VectorSubcoreMesh(core_axis_name="c", subcore_axis_name="s")
    mf = pl.kernel(
        functools.partial(_mf_body, b_per_w, d_half),
        out_type=jax.ShapeDtypeStruct((batch,), jnp.float32),
        mesh=mesh,
        compiler_params=pltpu.CompilerParams(
            needs_layout_passes=False, use_tc_tiling_on_sc=False),
        scratch_types=[
            pltpu.VMEM((b_per_w,), jnp.int32),
            pltpu.VMEM((b_per_w,), jnp.int32),
            pltpu.VMEM((b_per_w,), jnp.int32),
            pltpu.VMEM((b_per_w,), jnp.int32),
            pltpu.VMEM((2, _WAVE, _CHUNK), jnp.int32),
            pltpu.VMEM((2, _WAVE, _CHUNK), jnp.int32),
            pltpu.VMEM((b_per_w,), jnp.float32),
            pltpu.SemaphoreType.DMA,
        ],
    )
    return mf(idx_u.astype(jnp.int32), idx_i.astype(jnp.int32),
              _pack_table(embeds_u), _pack_table(embeds_i))


# speedup vs baseline: 6.8018x; 6.8018x over previous
import functools

import jax
import jax.numpy as jnp
from jax import lax
from jax.experimental import pallas as pl
from jax.experimental.pallas import tpu as pltpu
from jax.experimental.pallas import tpu_sc as plsc

_NC = 2
_NS = 16
_NW = _NC * _NS
_LANES = 16
_CHUNK = 128
_ROWS_PER_BLOCK = 8
_WAVE = 128


def _mf_body(b_per_w, d_half, idx_u_hbm, idx_i_hbm, pu_hbm, pi_hbm, out_hbm,
             idx_u_v, idx_i_v, blk_u_v, blk_i_v, u_rows, i_rows, out_v, sem):
    wid = lax.axis_index("s") * _NC + lax.axis_index("c")
    base = wid * b_per_w
    n_waves = b_per_w // _WAVE

    cp_u = pltpu.async_copy(idx_u_hbm.at[pl.ds(base, b_per_w)], idx_u_v, sem)
    cp_i = pltpu.async_copy(idx_i_hbm.at[pl.ds(base, b_per_w)], idx_i_v, sem)
    cp_u.wait()
    cp_i.wait()

    def blockify(t, carry):
        sl = pl.ds(t * _LANES, _LANES)
        blk_u_v[sl] = jax.lax.shift_right_logical(idx_u_v[sl], 3)
        blk_i_v[sl] = jax.lax.shift_right_logical(idx_i_v[sl], 3)
        return carry

    lax.fori_loop(0, b_per_w // _LANES, blockify, None)

    def fire(w):
        copies = []
        for k in range(_WAVE // _CHUNK):
            sl = pl.ds(w * _WAVE + k * _CHUNK, _CHUNK)
            dsl = pl.ds(k * _CHUNK, _CHUNK)
            copies.append(pltpu.async_copy(
                pu_hbm.at[blk_u_v.at[sl]], u_rows.at[w % 2].at[dsl], sem))
            copies.append(pltpu.async_copy(
                pi_hbm.at[blk_i_v.at[sl]], i_rows.at[w % 2].at[dsl], sem))
        return copies

    def compute(w):
        ubuf = u_rows.at[w % 2]
        ibuf = i_rows.at[w % 2]

        def group(g, carry):
            sl = pl.ds(w * _WAVE + g * _LANES, _LANES)
            row = g * _LANES + lax.iota(jnp.int32, _LANES)
            off_u = jax.lax.shift_left(idx_u_v[sl] & 7, 4)
            off_i = jax.lax.shift_left(idx_i_v[sl] & 7, 4)
            acc = jnp.zeros((_LANES,), jnp.float32)
            for j in range(d_half):
                gu = plsc.load_gather(ubuf, [row, off_u + j])
                gi = plsc.load_gather(ibuf, [row, off_i + j])
                u0, u1 = plsc.unpack(plsc.bitcast(gu, jnp.bfloat16),
                                     format=plsc.PackFormat.INTERLEAVED)
                i0, i1 = plsc.unpack(plsc.bitcast(gi, jnp.bfloat16),
                                     format=plsc.PackFormat.INTERLEAVED)
                acc = acc + u0 * i0 + u1 * i1
            out_v[sl] = 1.0 / (1.0 + jnp.exp(-acc))
            return carry

        lax.fori_loop(0, _WAVE // _LANES, group, None)

    pending = fire(0)
    for w in range(n_waves):
        for cp in pending:
            cp.wait()
        if w + 1 < n_waves:
            pending = fire(w + 1)
        compute(w)

    pltpu.sync_copy(out_v, out_hbm.at[pl.ds(base, b_per_w)])


def _pack_table(embeds):
    n, d = embeds.shape
    rows = (n * d) // (2 * _CHUNK)
    x = jax.lax.bitcast_convert_type(embeds, jnp.int32)
    r = jax.lax.shift_right_logical(
        x + 0x7FFF + (jax.lax.shift_right_logical(x, 16) & 1), 16)
    packed = (r[:, 0::2] & 0xFFFF) | jax.lax.shift_left(r[:, 1::2], 16)
    return packed.reshape(rows, _CHUNK)


def kernel(idx_u, idx_i, embeds_u, embeds_i):
    batch = idx_u.shape[0]
    d_half = embeds_u.shape[1] // 2
    b_per_w = batch // _NW
    mesh = plsc.VectorSubcoreMesh(core_axis_name="c", subcore_axis_name="s")
    mf = pl.kernel(
        functools.partial(_mf_body, b_per_w, d_half),
        out_type=jax.ShapeDtypeStruct((batch,), jnp.float32),
        mesh=mesh,
        compiler_params=pltpu.CompilerParams(
            needs_layout_passes=False, use_tc_tiling_on_sc=False),
        scratch_types=[
            pltpu.VMEM((b_per_w,), jnp.int32),
            pltpu.VMEM((b_per_w,), jnp.int32),
            pltpu.VMEM((b_per_w,), jnp.int32),
            pltpu.VMEM((b_per_w,), jnp.int32),
            pltpu.VMEM((2, _WAVE, _CHUNK), jnp.int32),
            pltpu.VMEM((2, _WAVE, _CHUNK), jnp.int32),
            pltpu.VMEM((b_per_w,), jnp.float32),
            pltpu.SemaphoreType.DMA,
        ],
    )
    return mf(idx_u.astype(jnp.int32), idx_i.astype(jnp.int32),
              _pack_table(embeds_u), _pack_table(embeds_i))


# speedup vs baseline: 14.6849x; 2.1590x over previous
import functools

import jax
import jax.numpy as jnp
from jax import lax
from jax.experimental import pallas as pl
from jax.experimental.pallas import tpu as pltpu
from jax.experimental.pallas import tpu_sc as plsc

_NC = 2
_NS = 16
_NW = _NC * _NS
_LANES = 16
_CHUNK = 128


def _mf_body(b_per_w, d_latent, idx_u_hbm, idx_i_hbm, bu_hbm, bi_hbm, out_hbm,
             idx_u_v, idx_i_v, u_bf, i_bf, u_pk, i_pk, out_v, sem):
    d_half = d_latent // 2
    wid = lax.axis_index("s") * _NC + lax.axis_index("c")
    base = wid * b_per_w

    cp_u = pltpu.async_copy(idx_u_hbm.at[pl.ds(base, b_per_w)], idx_u_v, sem)
    cp_i = pltpu.async_copy(idx_i_hbm.at[pl.ds(base, b_per_w)], idx_i_v, sem)
    cp_u.wait()
    cp_i.wait()

    copies = []
    for k in range(b_per_w // _CHUNK):
        sl = pl.ds(k * _CHUNK, _CHUNK)
        copies.append(pltpu.async_copy(bu_hbm.at[idx_u_v.at[sl]], u_bf.at[sl], sem))
        copies.append(pltpu.async_copy(bi_hbm.at[idx_i_v.at[sl]], i_bf.at[sl], sem))
    for cp in copies:
        cp.wait()

    def repack(e, carry):
        u_pk[e, :] = plsc.bitcast(u_bf[e, :], jnp.int32)
        i_pk[e, :] = plsc.bitcast(i_bf[e, :], jnp.int32)
        return carry

    lax.fori_loop(0, b_per_w, repack, None)

    def group(g, carry):
        row = g * _LANES + lax.iota(jnp.int32, _LANES)
        acc = jnp.zeros((_LANES,), jnp.float32)
        for j in range(d_half):
            col = jnp.full((_LANES,), j, jnp.int32)
            gu = plsc.load_gather(u_pk, [row, col])
            gi = plsc.load_gather(i_pk, [row, col])
            u0, u1 = plsc.unpack(plsc.bitcast(gu, jnp.bfloat16),
                                 format=plsc.PackFormat.INTERLEAVED)
            i0, i1 = plsc.unpack(plsc.bitcast(gi, jnp.bfloat16),
                                 format=plsc.PackFormat.INTERLEAVED)
            acc = acc + u0 * i0 + u1 * i1
        sig = 1.0 / (1.0 + jnp.exp(-acc))
        out_v[pl.ds(g * _LANES, _LANES)] = sig
        return carry

    lax.fori_loop(0, b_per_w // _LANES, group, None)
    pltpu.sync_copy(out_v, out_hbm.at[pl.ds(base, b_per_w)])


def kernel(idx_u, idx_i, embeds_u, embeds_i):
    batch = idx_u.shape[0]
    d_latent = embeds_u.shape[1]
    b_per_w = batch // _NW
    mesh = plsc.VectorSubcoreMesh(core_axis_name="c", subcore_axis_name="s")
    mf = pl.kernel(
        functools.partial(_mf_body, b_per_w, d_latent),
        out_type=jax.ShapeDtypeStruct((batch,), jnp.float32),
        mesh=mesh,
        compiler_params=pltpu.CompilerParams(
            needs_layout_passes=False, use_tc_tiling_on_sc=False),
        scratch_types=[
            pltpu.VMEM((b_per_w,), jnp.int32),
            pltpu.VMEM((b_per_w,), jnp.int32),
            pltpu.VMEM((b_per_w, d_latent), jnp.bfloat16),
            pltpu.VMEM((b_per_w, d_latent), jnp.bfloat16),
            pltpu.VMEM((b_per_w, d_latent // 2), jnp.int32),
            pltpu.VMEM((b_per_w, d_latent // 2), jnp.int32),
            pltpu.VMEM((b_per_w,), jnp.float32),
            pltpu.SemaphoreType.DMA,
        ],
    )
    return mf(idx_u.astype(jnp.int32), idx_i.astype(jnp.int32),
              embeds_u.astype(jnp.bfloat16), embeds_i.astype(jnp.bfloat16))


# speedup vs baseline: 16.9607x; 1.1550x over previous
import functools

import jax
import jax.numpy as jnp
from jax import lax
from jax.experimental import pallas as pl
from jax.experimental.pallas import tpu as pltpu
from jax.experimental.pallas import tpu_sc as plsc

_NC = 2
_NS = 16
_NW = _NC * _NS
_LANES = 16
_CHUNK = 128


def _mf_body(b_per_w, d_latent, idx_u_hbm, idx_i_hbm, eu_hbm, ei_hbm, out_hbm,
             idx_u_v, idx_i_v, u_rows, i_rows, out_v, sem):
    wid = lax.axis_index("s") * _NC + lax.axis_index("c")
    base = wid * b_per_w

    cp_u = pltpu.async_copy(idx_u_hbm.at[pl.ds(base, b_per_w)], idx_u_v, sem)
    cp_i = pltpu.async_copy(idx_i_hbm.at[pl.ds(base, b_per_w)], idx_i_v, sem)
    cp_u.wait()
    cp_i.wait()

    copies = []
    for k in range(b_per_w // _CHUNK):
        sl = pl.ds(k * _CHUNK, _CHUNK)
        copies.append(pltpu.async_copy(eu_hbm.at[idx_u_v.at[sl]], u_rows.at[sl], sem))
        copies.append(pltpu.async_copy(ei_hbm.at[idx_i_v.at[sl]], i_rows.at[sl], sem))
    for cp in copies:
        cp.wait()

    def group(g, carry):
        row = g * _LANES + lax.iota(jnp.int32, _LANES)
        acc = jnp.zeros((_LANES,), jnp.float32)
        for j in range(d_latent):
            col = jnp.full((_LANES,), j, jnp.int32)
            cu = plsc.load_gather(u_rows, [row, col])
            ci = plsc.load_gather(i_rows, [row, col])
            acc = acc + cu * ci
        sig = 1.0 / (1.0 + jnp.exp(-acc))
        out_v[pl.ds(g * _LANES, _LANES)] = sig
        return carry

    lax.fori_loop(0, b_per_w // _LANES, group, None)
    pltpu.sync_copy(out_v, out_hbm.at[pl.ds(base, b_per_w)])


def kernel(idx_u, idx_i, embeds_u, embeds_i):
    batch = idx_u.shape[0]
    d_latent = embeds_u.shape[1]
    b_per_w = batch // _NW
    mesh = plsc.VectorSubcoreMesh(core_axis_name="c", subcore_axis_name="s")
    mf = pl.kernel(
        functools.partial(_mf_body, b_per_w, d_latent),
        out_type=jax.ShapeDtypeStruct((batch,), jnp.float32),
        mesh=mesh,
        compiler_params=pltpu.CompilerParams(
            needs_layout_passes=False, use_tc_tiling_on_sc=False),
        scratch_types=[
            pltpu.VMEM((b_per_w,), jnp.int32),
            pltpu.VMEM((b_per_w,), jnp.int32),
            pltpu.VMEM((b_per_w, d_latent), jnp.float32),
            pltpu.VMEM((b_per_w, d_latent), jnp.float32),
            pltpu.VMEM((b_per_w,), jnp.float32),
            pltpu.SemaphoreType.DMA,
        ],
    )
    return mf(idx_u.astype(jnp.int32), idx_i.astype(jnp.int32), embeds_u, embeds_i)


# speedup vs baseline: 16.9794x; 1.0011x over previous
import functools

import jax
import jax.numpy as jnp
from jax import lax
from jax.experimental import pallas as pl
from jax.experimental.pallas import tpu as pltpu
from jax.experimental.pallas import tpu_sc as plsc

_NC = 2
_NS = 16
_NW = _NC * _NS
_LANES = 16
_CHUNK = 128
_BLOCK = 128
_WAVE = 128


def _mf_body(b_per_w, d_latent, idx_u_hbm, idx_i_hbm, pu_hbm, pi_hbm, out_hbm,
             idx_u_v, idx_i_v, blk_u_v, blk_i_v, u_rows, i_rows, out_v, sem):
    rows_per_block = _BLOCK // d_latent
    shift = rows_per_block.bit_length() - 1
    wid = lax.axis_index("s") * _NC + lax.axis_index("c")
    base = wid * b_per_w
    n_waves = b_per_w // _WAVE

    cp_u = pltpu.async_copy(idx_u_hbm.at[pl.ds(base, b_per_w)], idx_u_v, sem)
    cp_i = pltpu.async_copy(idx_i_hbm.at[pl.ds(base, b_per_w)], idx_i_v, sem)
    cp_u.wait()
    cp_i.wait()

    def blockify(t, carry):
        sl = pl.ds(t * _LANES, _LANES)
        blk_u_v[sl] = jax.lax.shift_right_logical(idx_u_v[sl], shift)
        blk_i_v[sl] = jax.lax.shift_right_logical(idx_i_v[sl], shift)
        return carry

    lax.fori_loop(0, b_per_w // _LANES, blockify, None)

    def fire(w):
        copies = []
        for k in range(_WAVE // _CHUNK):
            sl = pl.ds(w * _WAVE + k * _CHUNK, _CHUNK)
            dsl = pl.ds(k * _CHUNK, _CHUNK)
            copies.append(pltpu.async_copy(
                pu_hbm.at[blk_u_v.at[sl]], u_rows.at[w % 2].at[dsl], sem))
            copies.append(pltpu.async_copy(
                pi_hbm.at[blk_i_v.at[sl]], i_rows.at[w % 2].at[dsl], sem))
        return copies

    def compute(w):
        ubuf = u_rows.at[w % 2]
        ibuf = i_rows.at[w % 2]

        def group(g, carry):
            sl = pl.ds(w * _WAVE + g * _LANES, _LANES)
            row = g * _LANES + lax.iota(jnp.int32, _LANES)
            mask = rows_per_block - 1
            off_u = (idx_u_v[sl] & mask) * d_latent
            off_i = (idx_i_v[sl] & mask) * d_latent
            acc = jnp.zeros((_LANES,), jnp.float32)
            for j in range(d_latent):
                cu = plsc.load_gather(ubuf, [row, off_u + j])
                ci = plsc.load_gather(ibuf, [row, off_i + j])
                acc = acc + cu * ci
            out_v[sl] = 1.0 / (1.0 + jnp.exp(-acc))
            return carry

        lax.fori_loop(0, _WAVE // _LANES, group, None)

    pending = fire(0)
    for w in range(n_waves):
        for cp in pending:
            cp.wait()
        if w + 1 < n_waves:
            pending = fire(w + 1)
        compute(w)

    pltpu.sync_copy(out_v, out_hbm.at[pl.ds(base, b_per_w)])


def kernel(idx_u, idx_i, embeds_u, embeds_i):
    batch = idx_u.shape[0]
    n, d_latent = embeds_u.shape
    b_per_w = batch // _NW
    blocks = (n * d_latent) // _BLOCK
    mesh = plsc.VectorSubcoreMesh(core_axis_name="c", subcore_axis_name="s")
    mf = pl.kernel(
        functools.partial(_mf_body, b_per_w, d_latent),
        out_type=jax.ShapeDtypeStruct((batch,), jnp.float32),
        mesh=mesh,
        compiler_params=pltpu.CompilerParams(
            needs_layout_passes=False, use_tc_tiling_on_sc=False),
        scratch_types=[
            pltpu.VMEM((b_per_w,), jnp.int32),
            pltpu.VMEM((b_per_w,), jnp.int32),
            pltpu.VMEM((b_per_w,), jnp.int32),
            pltpu.VMEM((b_per_w,), jnp.int32),
            pltpu.VMEM((2, _WAVE, _BLOCK), jnp.float32),
            pltpu.VMEM((2, _WAVE, _BLOCK), jnp.float32),
            pltpu.VMEM((b_per_w,), jnp.float32),
            pltpu.SemaphoreType.DMA,
        ],
    )
    return mf(idx_u.astype(jnp.int32), idx_i.astype(jnp.int32),
              embeds_u.reshape(blocks, _BLOCK), embeds_i.reshape(blocks, _BLOCK))


# speedup vs baseline: 16.9882x; 1.0005x over previous
import functools

import jax
import jax.numpy as jnp
from jax import lax
from jax.experimental import pallas as pl
from jax.experimental.pallas import tpu as pltpu
from jax.experimental.pallas import tpu_sc as plsc

_NC = 2
_NS = 16
_NW = _NC * _NS
_LANES = 16
_CHUNK = 128
_BLOCK = 128
_WAVE = 128


def _mf_body(b_per_w, d_latent, idx_u_hbm, idx_i_hbm, pu_hbm, pi_hbm, out_hbm,
             idx_u_v, idx_i_v, blk_u_v, blk_i_v, u_rows, i_rows, out_v, sem):
    rows_per_block = _BLOCK // d_latent
    shift = rows_per_block.bit_length() - 1
    wid = lax.axis_index("s") * _NC + lax.axis_index("c")
    base = wid * b_per_w
    n_waves = b_per_w // _WAVE

    cp_u = pltpu.async_copy(idx_u_hbm.at[pl.ds(base, b_per_w)], idx_u_v, sem)
    cp_i = pltpu.async_copy(idx_i_hbm.at[pl.ds(base, b_per_w)], idx_i_v, sem)
    cp_u.wait()
    cp_i.wait()

    def blockify(t, carry):
        sl = pl.ds(t * _LANES, _LANES)
        blk_u_v[sl] = jax.lax.shift_right_logical(idx_u_v[sl], shift)
        blk_i_v[sl] = jax.lax.shift_right_logical(idx_i_v[sl], shift)
        return carry

    lax.fori_loop(0, b_per_w // _LANES, blockify, None)

    def fire(w):
        copies = []
        for k in range(_WAVE // _CHUNK):
            sl = pl.ds(w * _WAVE + k * _CHUNK, _CHUNK)
            dsl = pl.ds(k * _CHUNK, _CHUNK)
            copies.append(pltpu.async_copy(
                pu_hbm.at[blk_u_v.at[sl]], u_rows.at[w % 2].at[dsl], sem))
            copies.append(pltpu.async_copy(
                pi_hbm.at[blk_i_v.at[sl]], i_rows.at[w % 2].at[dsl], sem))
        return copies

    def compute(w):
        ubuf = u_rows.at[w % 2]
        ibuf = i_rows.at[w % 2]

        def group(g, carry):
            sl = pl.ds(w * _WAVE + g * _LANES, _LANES)
            row = g * _LANES + lax.iota(jnp.int32, _LANES)
            mask = rows_per_block - 1
            off_u = (idx_u_v[sl] & mask) * d_latent
            off_i = (idx_i_v[sl] & mask) * d_latent
            acc = jnp.zeros((_LANES,), jnp.float32)
            for j in range(d_latent):
                cu = plsc.load_gather(ubuf, [row, off_u + j])
                ci = plsc.load_gather(ibuf, [row, off_i + j])
                acc = acc + cu * ci
            out_v[sl] = 1.0 / (1.0 + jnp.exp(-acc))
            return carry

        lax.fori_loop(0, _WAVE // _LANES, group, None)

    pending = fire(0)
    for w in range(n_waves):
        for cp in pending:
            cp.wait()
        if w + 1 < n_waves:
            pending = fire(w + 1)
        compute(w)

    pltpu.sync_copy(out_v, out_hbm.at[pl.ds(base, b_per_w)])


def kernel(idx_u, idx_i, embeds_u, embeds_i):
    batch = idx_u.shape[0]
    n, d_latent = embeds_u.shape
    b_per_w = batch // _NW
    blocks = (n * d_latent) // _BLOCK
    mesh = plsc.VectorSubcoreMesh(core_axis_name="c", subcore_axis_name="s")
    mf = pl.kernel(
        functools.partial(_mf_body, b_per_w, d_latent),
        out_type=jax.ShapeDtypeStruct((batch,), jnp.float32),
        mesh=mesh,
        compiler_params=pltpu.CompilerParams(
            needs_layout_passes=False, use_tc_tiling_on_sc=True),
        scratch_types=[
            pltpu.VMEM((b_per_w,), jnp.int32),
            pltpu.VMEM((b_per_w,), jnp.int32),
            pltpu.VMEM((b_per_w,), jnp.int32),
            pltpu.VMEM((b_per_w,), jnp.int32),
            pltpu.VMEM((2, _WAVE, _BLOCK), jnp.float32),
            pltpu.VMEM((2, _WAVE, _BLOCK), jnp.float32),
            pltpu.VMEM((b_per_w,), jnp.float32),
            pltpu.SemaphoreType.DMA,
        ],
    )
    return mf(idx_u.astype(jnp.int32), idx_i.astype(jnp.int32),
              embeds_u.reshape(blocks, _BLOCK), embeds_i.reshape(blocks, _BLOCK))
